# baseline (speedup 1.0000x reference)
import functools
import math

import jax
import jax.numpy as jnp
from jax import lax
from jax.experimental import pallas as pl
from jax.experimental.pallas import tpu as pltpu
from jax.experimental.pallas import tpu_sc as plsc

VOCAB = 1000000
D = 64
ROWS = 4096
COLS = 200
NC = 2
NS = 16
NW = NC * NS
SCALE = math.sqrt(D)

T_UNITS = 7813
T_STEPS = 245

CH = 128
RB = ROWS // CH
UNITS2 = COLS * RB
STEPS2 = UNITS2 // NW

_mesh = plsc.VectorSubcoreMesh(core_axis_name="c", subcore_axis_name="s")
_params = pltpu.CompilerParams(use_tc_tiling_on_sc=True, needs_layout_passes=False)


@functools.partial(
    pl.kernel,
    mesh=_mesh,
    out_type=jax.ShapeDtypeStruct((VOCAB // 2, 128), jnp.float32),
    compiler_params=_params,
    scratch_types=[
        pltpu.VMEM((2, 64, 128), jnp.float32),
        pltpu.VMEM((2, 64, 128), jnp.float32),
    ]
    + [pltpu.SemaphoreType.DMA] * 4,
)
def _transpose_table(tT_hbm, tail_hbm, t2_hbm, vin, vout, *sems):
    gsem = sems[:2]
    ssem = sems[2:]
    wid = lax.axis_index("s") * NC + lax.axis_index("c")
    iota = lax.iota(jnp.int32, 16)

    def unit_id(k):
        return k * NW + wid

    def start_read(u, p):
        @pl.when(u < T_UNITS - 1)
        def _():
            pltpu.async_copy(tT_hbm.at[:, pl.ds(u * 128, 128)], vin.at[p], gsem[p])

        @pl.when(u == T_UNITS - 1)
        def _():
            pltpu.async_copy(tail_hbm, vin.at[p], gsem[p])

    def wait_read(u, p):
        pltpu.make_async_copy(
            tT_hbm.at[:, pl.ds(0, 128)], vin.at[p], gsem[p]
        ).wait()

    def start_write(u, p):
        @pl.when(u < T_UNITS - 1)
        def _():
            pltpu.async_copy(vout.at[p], t2_hbm.at[pl.ds(u * 64, 64)], ssem[p])

        @pl.when(u == T_UNITS - 1)
        def _():
            pltpu.async_copy(
                vout.at[p, pl.ds(0, 32)],
                t2_hbm.at[pl.ds(u * 64, 32)],
                ssem[p],
            )

    def wait_write(u, p):
        @pl.when(u < T_UNITS - 1)
        def _():
            pltpu.make_async_copy(
                vout.at[p], t2_hbm.at[pl.ds(0, 64)], ssem[p]
            ).wait()

        @pl.when(u == T_UNITS - 1)
        def _():
            pltpu.make_async_copy(
                vout.at[p, pl.ds(0, 32)], t2_hbm.at[pl.ds(0, 32)], ssem[p]
            ).wait()

    for p in range(2):
        @pl.when(unit_id(p) < T_UNITS)
        def _():
            start_read(unit_id(p), p)

    def step(s, carry):
        for p in range(2):
            k = 2 * s + p
            u = unit_id(k)

            @pl.when(u < T_UNITS)
            def _():
                wait_read(u, p)

                @pl.when(k >= 2)
                def _():
                    wait_write(unit_id(k - 2), p)

                @pl.when(u < T_UNITS - 1)
                def _():
                    def qbody(q, c2):
                        for t in range(8):
                            dvec = (t % 4) * 16 + iota
                            colv = jnp.zeros((16,), jnp.int32) + (2 * q + t // 4)
                            v = plsc.load_gather(vin.at[p], [dvec, colv])
                            vout[p, q, pl.ds(t * 16, 16)] = v
                        return c2

                    lax.fori_loop(0, 64, qbody, 0, unroll=2)

                @pl.when(u == T_UNITS - 1)
                def _():
                    def qbody(q, c2):
                        for t in range(8):
                            vout[p, q, pl.ds(t * 16, 16)] = vin[
                                p, 2 * q + t // 4, pl.ds((t % 4) * 16, 16)
                            ]
                        return c2

                    lax.fori_loop(0, 32, qbody, 0, unroll=2)

                start_write(u, p)
                nu = unit_id(k + 2)

                @pl.when(nu < T_UNITS)
                def _():
                    start_read(nu, p)

        return carry

    lax.fori_loop(0, (T_STEPS + 1) // 2, step, 0, unroll=False)
    for lastk in (T_STEPS - 3, T_STEPS - 2, T_STEPS - 1):
        @pl.when(
            (unit_id(lastk) < T_UNITS) & (unit_id(lastk + 2) >= T_UNITS)
        )
        def _():
            wait_write(unit_id(lastk), lastk % 2)


@functools.partial(
    pl.kernel,
    mesh=_mesh,
    out_type=jax.ShapeDtypeStruct((COLS * D, ROWS), jnp.float32),
    compiler_params=_params,
    scratch_types=[
        pltpu.VMEM((2, CH), jnp.int32),
        pltpu.VMEM((2, CH), jnp.int32),
        pltpu.VMEM((2, CH, 128), jnp.float32),
        pltpu.VMEM((2, D, CH), jnp.float32),
    ]
    + [pltpu.SemaphoreType.DMA] * 4,
)
def _lookup(xT_hbm, t2_hbm, out_hbm, idx_v, idx2_v, gbuf, tbuf, *sems):
    gsem = sems[:2]
    ssem = sems[2:]
    wid = lax.axis_index("s") * NC + lax.axis_index("c")
    iota = lax.iota(jnp.int32, 16)

    def prep_and_fire(u, p):
        c = u // RB
        rb = u - c * RB
        pltpu.sync_copy(xT_hbm.at[c, pl.ds(rb * CH, CH)], idx_v.at[p])
        for g in range(CH // 16):
            sl = pl.ds(g * 16, 16)
            idx2_v[p, sl] = lax.shift_right_logical(idx_v[p, sl], 1)
        pltpu.async_copy(t2_hbm.at[idx2_v.at[p]], gbuf.at[p], gsem[p])

    for p in range(2):
        prep_and_fire(wid * STEPS2 + p, p)

    def step(s, carry):
        for p in range(2):
            k = 2 * s + p
            u = wid * STEPS2 + k
            c = u // RB
            rb = u - c * RB
            pltpu.make_async_copy(
                t2_hbm.at[idx2_v.at[p]], gbuf.at[p], gsem[p]
            ).wait()

            @pl.when(k >= 2)
            def _():
                pltpu.make_async_copy(
                    tbuf.at[p], out_hbm.at[pl.ds(0, D), pl.ds(0, CH)], ssem[p]
                ).wait()

            for g in range(CH // 16):
                sl = pl.ds(g * 16, 16)
                rows16 = g * 16 + iota
                p64 = (idx_v[p, sl] & 1) * 64

                def dbody(d, c2):
                    v = plsc.load_gather(gbuf.at[p], [rows16, p64 + d])
                    tbuf[p, d, sl] = v * SCALE
                    return c2

                lax.fori_loop(0, D, dbody, 0, unroll=4)
            pltpu.async_copy(
                tbuf.at[p],
                out_hbm.at[pl.ds(c * D, D), pl.ds(rb * CH, CH)],
                ssem[p],
            )

            @pl.when(k + 2 < STEPS2)
            def _():
                prep_and_fire(wid * STEPS2 + k + 2, p)

        return carry

    lax.fori_loop(0, STEPS2 // 2, step, 0, unroll=False)
    for p in range(2):
        pltpu.make_async_copy(
            tbuf.at[p], out_hbm.at[pl.ds(0, D), pl.ds(0, CH)], ssem[p]
        ).wait()


def kernel(x, table):
    xT = jnp.transpose(x).astype(jnp.int32)
    tT = jnp.transpose(table)
    tail = jnp.pad(table[VOCAB - 64 :, :], ((0, 0), (0, 64)))
    t2 = _transpose_table(tT, tail)
    out2 = _lookup(xT, t2)
    return jnp.transpose(out2.reshape(COLS, D, ROWS), (2, 0, 1))


# speedup vs baseline: 1.0024x; 1.0024x over previous
import functools
import math

import jax
import jax.numpy as jnp
from jax import lax
from jax.experimental import pallas as pl
from jax.experimental.pallas import tpu as pltpu
from jax.experimental.pallas import tpu_sc as plsc

VOCAB = 1000000
D = 64
ROWS = 4096
COLS = 200
NC = 2
NS = 16
NW = NC * NS
SCALE = math.sqrt(D)

T_UNITS = 7813
T_STEPS = 245

CH = 128
RB = ROWS // CH
UNITS2 = COLS * RB
STEPS2 = UNITS2 // NW

_mesh = plsc.VectorSubcoreMesh(core_axis_name="c", subcore_axis_name="s")
_params = pltpu.CompilerParams(
    use_tc_tiling_on_sc=True,
    needs_layout_passes=False,
    disable_bounds_checks=True,
)


@functools.partial(
    pl.kernel,
    mesh=_mesh,
    out_type=jax.ShapeDtypeStruct((VOCAB // 2, 128), jnp.float32),
    compiler_params=_params,
    scratch_types=[
        pltpu.VMEM((2, 64, 128), jnp.float32),
        pltpu.VMEM((2, 64, 128), jnp.float32),
    ]
    + [pltpu.SemaphoreType.DMA] * 4,
)
def _transpose_table(tT_hbm, tail_hbm, t2_hbm, vin, vout, *sems):
    gsem = sems[:2]
    ssem = sems[2:]
    wid = lax.axis_index("s") * NC + lax.axis_index("c")
    iota = lax.iota(jnp.int32, 16)

    def unit_id(k):
        return k * NW + wid

    def start_read(u, p):
        @pl.when(u < T_UNITS - 1)
        def _():
            pltpu.async_copy(tT_hbm.at[:, pl.ds(u * 128, 128)], vin.at[p], gsem[p])

        @pl.when(u == T_UNITS - 1)
        def _():
            pltpu.async_copy(tail_hbm, vin.at[p], gsem[p])

    def wait_read(u, p):
        pltpu.make_async_copy(
            tT_hbm.at[:, pl.ds(0, 128)], vin.at[p], gsem[p]
        ).wait()

    def start_write(u, p):
        @pl.when(u < T_UNITS - 1)
        def _():
            pltpu.async_copy(vout.at[p], t2_hbm.at[pl.ds(u * 64, 64)], ssem[p])

        @pl.when(u == T_UNITS - 1)
        def _():
            pltpu.async_copy(
                vout.at[p, pl.ds(0, 32)],
                t2_hbm.at[pl.ds(u * 64, 32)],
                ssem[p],
            )

    def wait_write(u, p):
        @pl.when(u < T_UNITS - 1)
        def _():
            pltpu.make_async_copy(
                vout.at[p], t2_hbm.at[pl.ds(0, 64)], ssem[p]
            ).wait()

        @pl.when(u == T_UNITS - 1)
        def _():
            pltpu.make_async_copy(
                vout.at[p, pl.ds(0, 32)], t2_hbm.at[pl.ds(0, 32)], ssem[p]
            ).wait()

    for p in range(2):
        @pl.when(unit_id(p) < T_UNITS)
        def _():
            start_read(unit_id(p), p)

    def step(s, carry):
        for p in range(2):
            k = 2 * s + p
            u = unit_id(k)

            @pl.when(u < T_UNITS)
            def _():
                wait_read(u, p)

                @pl.when(k >= 2)
                def _():
                    wait_write(unit_id(k - 2), p)

                @pl.when(u < T_UNITS - 1)
                def _():
                    dvecs = [t * 16 + iota for t in range(4)]

                    def qbody(q, c2):
                        colA = jnp.zeros((16,), jnp.int32) + 2 * q
                        colB = colA + 1
                        for t in range(8):
                            colv = colA if t < 4 else colB
                            v = plsc.load_gather(vin.at[p], [dvecs[t % 4], colv])
                            vout[p, q, pl.ds(t * 16, 16)] = v
                        return c2

                    lax.fori_loop(0, 64, qbody, 0, unroll=2)

                @pl.when(u == T_UNITS - 1)
                def _():
                    def qbody(q, c2):
                        for t in range(8):
                            vout[p, q, pl.ds(t * 16, 16)] = vin[
                                p, 2 * q + t // 4, pl.ds((t % 4) * 16, 16)
                            ]
                        return c2

                    lax.fori_loop(0, 32, qbody, 0, unroll=2)

                start_write(u, p)
                nu = unit_id(k + 2)

                @pl.when(nu < T_UNITS)
                def _():
                    start_read(nu, p)

        return carry

    lax.fori_loop(0, (T_STEPS + 1) // 2, step, 0, unroll=False)
    for lastk in (T_STEPS - 3, T_STEPS - 2, T_STEPS - 1):
        @pl.when(
            (unit_id(lastk) < T_UNITS) & (unit_id(lastk + 2) >= T_UNITS)
        )
        def _():
            wait_write(unit_id(lastk), lastk % 2)


@functools.partial(
    pl.kernel,
    mesh=_mesh,
    out_type=jax.ShapeDtypeStruct((COLS * D, ROWS), jnp.float32),
    compiler_params=_params,
    scratch_types=[
        pltpu.VMEM((2, CH), jnp.int32),
        pltpu.VMEM((2, CH), jnp.int32),
        pltpu.VMEM((2, CH, 128), jnp.float32),
        pltpu.VMEM((2, D, CH), jnp.float32),
    ]
    + [pltpu.SemaphoreType.DMA] * 4,
)
def _lookup(xT_hbm, t2_hbm, out_hbm, idx_v, idx2_v, gbuf, tbuf, *sems):
    gsem = sems[:2]
    ssem = sems[2:]
    wid = lax.axis_index("s") * NC + lax.axis_index("c")
    iota = lax.iota(jnp.int32, 16)

    def prep_and_fire(u, p):
        c = u // RB
        rb = u - c * RB
        pltpu.sync_copy(xT_hbm.at[c, pl.ds(rb * CH, CH)], idx_v.at[p])
        for g in range(CH // 16):
            sl = pl.ds(g * 16, 16)
            idx2_v[p, sl] = lax.shift_right_logical(idx_v[p, sl], 1)
        pltpu.async_copy(t2_hbm.at[idx2_v.at[p]], gbuf.at[p], gsem[p])

    for p in range(2):
        prep_and_fire(wid * STEPS2 + p, p)

    def step(s, carry):
        for p in range(2):
            k = 2 * s + p
            u = wid * STEPS2 + k
            c = u // RB
            rb = u - c * RB
            pltpu.make_async_copy(
                t2_hbm.at[idx2_v.at[p]], gbuf.at[p], gsem[p]
            ).wait()

            @pl.when(k >= 2)
            def _():
                pltpu.make_async_copy(
                    tbuf.at[p], out_hbm.at[pl.ds(0, D), pl.ds(0, CH)], ssem[p]
                ).wait()

            rows16s = [g * 16 + iota for g in range(CH // 16)]
            p64s = [
                (idx_v[p, pl.ds(g * 16, 16)] & 1) << 6 for g in range(CH // 16)
            ]

            def dbody(d, c2):
                for g in range(CH // 16):
                    v = plsc.load_gather(gbuf.at[p], [rows16s[g], p64s[g] + d])
                    tbuf[p, d, pl.ds(g * 16, 16)] = v * SCALE
                return c2

            lax.fori_loop(0, D, dbody, 0, unroll=2)
            pltpu.async_copy(
                tbuf.at[p],
                out_hbm.at[pl.ds(c * D, D), pl.ds(rb * CH, CH)],
                ssem[p],
            )

            @pl.when(k + 2 < STEPS2)
            def _():
                prep_and_fire(wid * STEPS2 + k + 2, p)

        return carry

    lax.fori_loop(0, STEPS2 // 2, step, 0, unroll=False)
    for p in range(2):
        pltpu.make_async_copy(
            tbuf.at[p], out_hbm.at[pl.ds(0, D), pl.ds(0, CH)], ssem[p]
        ).wait()


def kernel(x, table):
    xT = jnp.transpose(x).astype(jnp.int32)
    tT = jnp.transpose(table)
    tail = jnp.pad(table[VOCAB - 64 :, :], ((0, 0), (0, 64)))
    t2 = _transpose_table(tT, tail)
    out2 = _lookup(xT, t2)
    return jnp.transpose(out2.reshape(COLS, D, ROWS), (2, 0, 1))


# speedup vs baseline: 1.0339x; 1.0314x over previous
import functools
import math

import jax
import jax.numpy as jnp
from jax import lax
from jax.experimental import pallas as pl
from jax.experimental.pallas import tpu as pltpu
from jax.experimental.pallas import tpu_sc as plsc

VOCAB = 1000000
D = 64
ROWS = 4096
COLS = 200
NC = 2
NS = 16
NW = NC * NS
SCALE = math.sqrt(D)

W1 = 256
T_UNITS = 3907
T_STEPS = 123

CH = 128
STEPS2 = COLS

_mesh = plsc.VectorSubcoreMesh(core_axis_name="c", subcore_axis_name="s")
_params = pltpu.CompilerParams(
    use_tc_tiling_on_sc=True,
    needs_layout_passes=False,
    disable_bounds_checks=True,
)


@functools.partial(
    pl.kernel,
    mesh=_mesh,
    out_type=jax.ShapeDtypeStruct((VOCAB // 2, 128), jnp.float32),
    compiler_params=_params,
    scratch_types=[
        pltpu.VMEM((4, 64, W1), jnp.float32),
        pltpu.VMEM((2, W1 // 2, 128), jnp.float32),
    ]
    + [pltpu.SemaphoreType.DMA] * 6,
)
def _transpose_table(tT_hbm, tail_hbm, t2_hbm, vin, vout, *sems):
    gsem = sems[:4]
    ssem = sems[4:]
    wid = lax.axis_index("s") * NC + lax.axis_index("c")
    iota = lax.iota(jnp.int32, 16)
    dvecs = [t * 16 + iota for t in range(4)]

    def unit_id(k):
        return k * NW + wid

    def valid(k):
        return unit_id(k) < T_UNITS

    def start_read(k, p):
        u = unit_id(k)

        @pl.when(u < T_UNITS - 1)
        def _():
            pltpu.async_copy(tT_hbm.at[:, pl.ds(u * W1, W1)], vin.at[p], gsem[p])

        @pl.when(u == T_UNITS - 1)
        def _():
            pltpu.async_copy(tail_hbm, vin.at[p, :, pl.ds(0, 128)], gsem[p])

    def wait_read(k, p):
        u = unit_id(k)

        @pl.when(u < T_UNITS - 1)
        def _():
            pltpu.make_async_copy(
                tT_hbm.at[:, pl.ds(0, W1)], vin.at[p], gsem[p]
            ).wait()

        @pl.when(u == T_UNITS - 1)
        def _():
            pltpu.make_async_copy(
                tT_hbm.at[:, pl.ds(0, 128)], vin.at[p, :, pl.ds(0, 128)], gsem[p]
            ).wait()

    def start_write(k, p):
        u = unit_id(k)

        @pl.when(u < T_UNITS - 1)
        def _():
            pltpu.async_copy(
                vout.at[p], t2_hbm.at[pl.ds(u * (W1 // 2), W1 // 2)], ssem[p]
            )

        @pl.when(u == T_UNITS - 1)
        def _():
            pltpu.async_copy(
                vout.at[p, pl.ds(0, 32)],
                t2_hbm.at[pl.ds(u * (W1 // 2), 32)],
                ssem[p],
            )

    def wait_write(k, p):
        u = unit_id(k)

        @pl.when(u < T_UNITS - 1)
        def _():
            pltpu.make_async_copy(
                vout.at[p], t2_hbm.at[pl.ds(0, W1 // 2)], ssem[p]
            ).wait()

        @pl.when(u == T_UNITS - 1)
        def _():
            pltpu.make_async_copy(
                vout.at[p, pl.ds(0, 32)], t2_hbm.at[pl.ds(0, 32)], ssem[p]
            ).wait()

    for k in range(3):
        @pl.when(valid(k))
        def _():
            start_read(k, k % 4)

    def step(s, carry):
        for j in range(4):
            k = 4 * s + j
            u = unit_id(k)
            p = j

            @pl.when(u < T_UNITS)
            def _():
                wait_read(k, j)

                @pl.when(valid(k + 3))
                def _():
                    start_read(k + 3, (j + 3) % 4)

                @pl.when(k >= 2)
                def _():
                    wait_write(k - 2, j % 2)

                @pl.when(u < T_UNITS - 1)
                def _():
                    def qbody(q, c2):
                        colA = jnp.zeros((16,), jnp.int32) + 2 * q
                        colB = colA + 1
                        for t in range(8):
                            colv = colA if t < 4 else colB
                            v = plsc.load_gather(vin.at[p], [dvecs[t % 4], colv])
                            vout[j % 2, q, pl.ds(t * 16, 16)] = v
                        return c2

                    lax.fori_loop(0, W1 // 2, qbody, 0, unroll=2)

                @pl.when(u == T_UNITS - 1)
                def _():
                    def qbody(q, c2):
                        for t in range(8):
                            vout[j % 2, q, pl.ds(t * 16, 16)] = vin[
                                p, 2 * q + t // 4, pl.ds((t % 4) * 16, 16)
                            ]
                        return c2

                    lax.fori_loop(0, 32, qbody, 0, unroll=2)

                start_write(k, j % 2)

        return carry

    lax.fori_loop(0, (T_STEPS + 3) // 4, step, 0, unroll=False)
    for lastk in (T_STEPS - 3, T_STEPS - 2, T_STEPS - 1):
        @pl.when(valid(lastk) & ~valid(lastk + 2))
        def _():
            wait_write(lastk, lastk % 2)


@functools.partial(
    pl.kernel,
    mesh=_mesh,
    out_type=jax.ShapeDtypeStruct((COLS * D, ROWS), jnp.float32),
    compiler_params=_params,
    scratch_types=[
        pltpu.VMEM((COLS, CH), jnp.int32),
        pltpu.VMEM((4, CH), jnp.int32),
        pltpu.VMEM((4, CH, 128), jnp.float32),
        pltpu.VMEM((2, D, CH), jnp.float32),
    ]
    + [pltpu.SemaphoreType.DMA] * 6,
)
def _lookup(xT_hbm, t2_hbm, out_hbm, idxall, idx2_v, gbuf, tbuf, *sems):
    gsem = sems[:4]
    ssem = sems[4:]
    wid = lax.axis_index("s") * NC + lax.axis_index("c")
    iota = lax.iota(jnp.int32, 16)
    rows16s = [g * 16 + iota for g in range(CH // 16)]

    pltpu.sync_copy(xT_hbm.at[:, pl.ds(wid * CH, CH)], idxall)

    def fire_gather(c, p):
        for g in range(CH // 16):
            sl = pl.ds(g * 16, 16)
            idx2_v[p, sl] = lax.shift_right_logical(idxall[c, sl], 1)
        pltpu.async_copy(t2_hbm.at[idx2_v.at[p]], gbuf.at[p], gsem[p])

    for c in range(3):
        fire_gather(c, c % 4)

    def step(s, carry):
        for j in range(4):
            c = 4 * s + j
            p = j
            w = j % 2
            pltpu.make_async_copy(
                t2_hbm.at[idx2_v.at[p]], gbuf.at[p], gsem[p]
            ).wait()

            @pl.when(c + 3 < STEPS2)
            def _():
                fire_gather(c + 3, (j + 3) % 4)

            @pl.when(c >= 2)
            def _():
                pltpu.make_async_copy(
                    tbuf.at[w], out_hbm.at[pl.ds(0, D), pl.ds(0, CH)], ssem[w]
                ).wait()

            p64s = [
                (idxall[c, pl.ds(g * 16, 16)] & 1) << 6 for g in range(CH // 16)
            ]

            def dbody(d, c2):
                for g in range(CH // 16):
                    v = plsc.load_gather(gbuf.at[p], [rows16s[g], p64s[g] + d])
                    tbuf[w, d, pl.ds(g * 16, 16)] = v * SCALE
                return c2

            lax.fori_loop(0, D, dbody, 0, unroll=2)
            pltpu.async_copy(
                tbuf.at[w],
                out_hbm.at[pl.ds(c * D, D), pl.ds(wid * CH, CH)],
                ssem[w],
            )

        return carry

    lax.fori_loop(0, STEPS2 // 4, step, 0, unroll=False)
    for w in range(2):
        pltpu.make_async_copy(
            tbuf.at[w], out_hbm.at[pl.ds(0, D), pl.ds(0, CH)], ssem[w]
        ).wait()


def kernel(x, table):
    xT = jnp.transpose(x).astype(jnp.int32)
    tT = jnp.transpose(table)
    tail = jnp.pad(table[VOCAB - 64 :, :], ((0, 0), (0, 64)))
    t2 = _transpose_table(tT, tail)
    out2 = _lookup(xT, t2)
    return jnp.transpose(out2.reshape(COLS, D, ROWS), (2, 0, 1))


# speedup vs baseline: 2.1665x; 2.0954x over previous
import functools
import math

import jax
import jax.numpy as jnp
from jax import lax
from jax.experimental import pallas as pl
from jax.experimental.pallas import tpu as pltpu
from jax.experimental.pallas import tpu_sc as plsc

VOCAB = 1000000
D = 64
ROWS = 4096
COLS = 200
B_TOT = ROWS * COLS
NC = 2
NS = 16
NW = NC * NS
B_PER_W = B_TOT // NW
CHUNK = 128
N_CHUNKS = B_PER_W // CHUNK
NBUF = 4
LANES = 16
SCALE = math.sqrt(D)

_mesh = plsc.VectorSubcoreMesh(core_axis_name="c", subcore_axis_name="s")


@functools.partial(
    pl.kernel,
    mesh=_mesh,
    out_type=jax.ShapeDtypeStruct((NW, N_CHUNKS, CHUNK, D), jnp.float32),
    compiler_params=pltpu.CompilerParams(use_tc_tiling_on_sc=False),
    scratch_types=[
        pltpu.VMEM((N_CHUNKS, CHUNK), jnp.int32),
        pltpu.VMEM((NBUF, CHUNK, D), jnp.float32),
        pltpu.VMEM((NBUF, CHUNK, D), jnp.float32),
    ]
    + [pltpu.SemaphoreType.DMA] * (2 * NBUF),
)
def _emb_lookup(x_hbm, table_hbm, out_hbm, idx_v, in_v, out_v, *sems):
    gsem = sems[:NBUF]
    ssem = sems[NBUF:]
    wid = lax.axis_index("s") * NC + lax.axis_index("c")
    pltpu.sync_copy(x_hbm.at[wid], idx_v)

    for b in range(NBUF):
        pltpu.async_copy(table_hbm.at[idx_v.at[b]], in_v.at[b], gsem[b])

    def group_body(g, carry):
        for b in range(NBUF):
            jj = g * NBUF + b
            pltpu.make_async_copy(
                table_hbm.at[idx_v.at[b]], in_v.at[b], gsem[b]
            ).wait()

            @pl.when(jj >= NBUF)
            def _():
                pltpu.make_async_copy(
                    out_v.at[b], out_hbm.at[wid, 0], ssem[b]
                ).wait()

            def row_body(i, c):
                for t in range(D // LANES):
                    sl = pl.ds(t * LANES, LANES)
                    out_v[b, i, sl] = in_v[b, i, sl] * SCALE
                return c

            lax.fori_loop(0, CHUNK, row_body, 0, unroll=8)
            pltpu.async_copy(out_v.at[b], out_hbm.at[wid, jj], ssem[b])

            @pl.when(jj + NBUF < N_CHUNKS)
            def _():
                pltpu.async_copy(
                    table_hbm.at[idx_v.at[jj + NBUF]], in_v.at[b], gsem[b]
                )

        return carry

    lax.fori_loop(0, N_CHUNKS // NBUF, group_body, 0)
    for b in range(NBUF):
        pltpu.make_async_copy(out_v.at[b], out_hbm.at[wid, 0], ssem[b]).wait()


def kernel(x, table):
    idx = x.astype(jnp.int32).reshape(NW, N_CHUNKS, CHUNK)
    out = _emb_lookup(idx, table)
    return out.reshape(ROWS, COLS, D)
